# Initial kernel scaffold; baseline (speedup 1.0000x reference)
import jax
import jax.numpy as jnp
from jax.experimental import pallas as pl


def kernel(cosine, label):
    raise NotImplementedError("write your pallas kernel here")


# speedup vs baseline: 2.5047x; 2.5047x over previous
import functools
import math

import jax
import jax.numpy as jnp
from jax.experimental import pallas as pl

S = 64.0
M = 0.5
COS_M = math.cos(M)
SIN_M = math.sin(M)

_BC = 2048


def _arcface_block(label_ref, cos_ref, out_ref):
    j = pl.program_id(0)
    c = cos_ref[...]
    lab = label_ref[...]
    col0 = j * _BC
    col_ids = col0 + jax.lax.broadcasted_iota(jnp.int32, c.shape, 1)
    is_target = lab == col_ids
    scaled = c * S
    penal = (c * COS_M - jnp.sqrt(jnp.maximum(1.0 - c * c, 0.0)) * SIN_M) * S
    out_ref[...] = jnp.where(is_target, penal, scaled)


def kernel(cosine, label):
    B, C = cosine.shape
    grid = (pl.cdiv(C, _BC),)
    label2d = label.reshape(B, 1)
    return pl.pallas_call(
        _arcface_block,
        grid=grid,
        in_specs=[
            pl.BlockSpec((B, 1), lambda j: (0, 0)),
            pl.BlockSpec((B, _BC), lambda j: (0, j)),
        ],
        out_specs=pl.BlockSpec((B, _BC), lambda j: (0, j)),
        out_shape=jax.ShapeDtypeStruct((B, C), cosine.dtype),
    )(label2d, cosine)
